# baseline (speedup 1.0000x reference)
import functools

import jax
import jax.numpy as jnp
from jax import lax
from jax.experimental import pallas as pl
from jax.experimental.pallas import tpu as pltpu
from jax.experimental.pallas import tpu_sc as plsc

_B = 16384
_DIM = 64
_MARGIN = 1.0

_NC = 2
_NS = 16
_NW = _NC * _NS

_CHUNK = 512
_GRP = 16


def _sc_gather(ent_idx, rel_idx, ent_flat, rel_emb):
    n_ent_rows = ent_idx.shape[0]
    n_rel_rows = rel_idx.shape[0]
    ent_per_w = n_ent_rows // _NW
    rel_per_w = n_rel_rows // _NW
    mesh = plsc.VectorSubcoreMesh(core_axis_name="c", subcore_axis_name="s")

    @functools.partial(
        pl.kernel,
        out_type=(
            jax.ShapeDtypeStruct((n_ent_rows * _DIM,), jnp.float32),
            jax.ShapeDtypeStruct((n_rel_rows, _DIM), jnp.float32),
        ),
        mesh=mesh,
        scratch_types=[
            pltpu.VMEM((_CHUNK,), jnp.int32),
            pltpu.VMEM((_CHUNK, _DIM), jnp.float32),
            pltpu.VMEM((_CHUNK * _DIM,), jnp.float32),
            pltpu.SemaphoreType.DMA,
        ],
    )
    def gather_kernel(ent_idx_hbm, rel_idx_hbm, ent_hbm, rel_hbm,
                      ent_out, rel_out, idx_v, rows_v, rowsf_v, sem):
        wid = lax.axis_index("s") * _NC + lax.axis_index("c")

        def do_flat_table(idx_hbm, tab_hbm, out_hbm, per_w):
            base = wid * per_w

            def chunk_body(c, carry):
                off = base + c * _CHUNK
                pltpu.sync_copy(idx_hbm.at[pl.ds(off, _CHUNK)], idx_v)

                def grp_fire(g, carry):
                    gbase = pl.multiple_of(g * _GRP, _GRP)
                    vec = idx_v[pl.ds(gbase, _GRP)]
                    for l in range(_GRP):
                        pltpu.make_async_copy(
                            tab_hbm.at[pl.ds(vec[l] * _DIM, _DIM)],
                            rowsf_v.at[pl.ds((gbase + l) * _DIM, _DIM)],
                            sem,
                        ).start()
                    return carry

                lax.fori_loop(0, _CHUNK // _GRP, grp_fire, jnp.int32(0))

                def grp_drain(g, carry):
                    pltpu.make_async_copy(
                        tab_hbm.at[pl.ds(0, _DIM)],
                        rowsf_v.at[pl.ds(0, _DIM)],
                        sem,
                    ).wait()
                    return carry

                lax.fori_loop(0, _CHUNK, grp_drain, jnp.int32(0))
                pltpu.sync_copy(
                    rowsf_v, out_hbm.at[pl.ds(off * _DIM, _CHUNK * _DIM)])
                return carry

            lax.fori_loop(0, per_w // _CHUNK, chunk_body, jnp.int32(0))

        def do_table(idx_hbm, tab_hbm, out_hbm, per_w):
            base = wid * per_w

            def chunk_body(c, carry):
                off = base + c * _CHUNK
                pltpu.sync_copy(idx_hbm.at[pl.ds(off, _CHUNK)], idx_v)

                def grp_fire(g, carry):
                    gbase = pl.multiple_of(g * _GRP, _GRP)
                    vec = idx_v[pl.ds(gbase, _GRP)]
                    for l in range(_GRP):
                        pltpu.make_async_copy(
                            tab_hbm.at[pl.ds(vec[l], 1), :],
                            rows_v.at[pl.ds(gbase + l, 1), :],
                            sem,
                        ).start()
                    return carry

                lax.fori_loop(0, _CHUNK // _GRP, grp_fire, jnp.int32(0))

                def grp_drain(g, carry):
                    pltpu.make_async_copy(
                        tab_hbm.at[pl.ds(0, 1), :],
                        rows_v.at[pl.ds(0, 1), :],
                        sem,
                    ).wait()
                    return carry

                lax.fori_loop(0, _CHUNK, grp_drain, jnp.int32(0))
                pltpu.sync_copy(rows_v, out_hbm.at[pl.ds(off, _CHUNK)])
                return carry

            lax.fori_loop(0, per_w // _CHUNK, chunk_body, jnp.int32(0))

        do_flat_table(ent_idx_hbm, ent_hbm, ent_out, ent_per_w)
        do_table(rel_idx_hbm, rel_hbm, rel_out, rel_per_w)

    return gather_kernel(ent_idx, rel_idx, ent_flat, rel_emb)


_BLK = 2048


def _dense_body(hp_ref, tp_ref, hn_ref, tn_ref, r_ref, out_ref):
    i = pl.program_id(0)

    def nrm(x):
        n = jnp.sqrt(jnp.sum(x * x, axis=1, keepdims=True))
        return x / jnp.maximum(n, 1e-12)

    hp = nrm(hp_ref[...])
    tp = nrm(tp_ref[...])
    hn = nrm(hn_ref[...])
    tn = nrm(tn_ref[...])
    r = nrm(r_ref[...])
    pos_e = jnp.sqrt(jnp.sum((hp + r - tp) ** 2, axis=1))
    neg_e = jnp.sqrt(jnp.sum((hn + r - tn) ** 2, axis=1))
    loss = jnp.maximum(_MARGIN + pos_e - neg_e, 0.0)
    s = jnp.sum(loss)

    @pl.when(i == 0)
    def _init():
        out_ref[0, 0] = s

    @pl.when(i != 0)
    def _acc():
        out_ref[0, 0] += s

    @pl.when(i == pl.num_programs(0) - 1)
    def _final():
        out_ref[0, 0] = out_ref[0, 0] / _B


def _dense_loss(ent_rows, rel_rows):
    grid = _B // _BLK
    nblk = grid

    def section(k):
        return pl.BlockSpec((_BLK, _DIM), lambda i, k=k: (k * nblk + i, 0))

    return pl.pallas_call(
        _dense_body,
        grid=(grid,),
        in_specs=[section(0), section(1), section(2), section(3),
                  pl.BlockSpec((_BLK, _DIM), lambda i: (i, 0))],
        out_specs=pl.BlockSpec((1, 1), lambda i: (0, 0),
                               memory_space=pltpu.SMEM),
        out_shape=jax.ShapeDtypeStruct((1, 1), jnp.float32),
    )(ent_rows, ent_rows, ent_rows, ent_rows, rel_rows)


def kernel(pos_pairs, neg_pairs, rels, ent_embs, alignments, rel_emb):
    ent_idx = jnp.concatenate(
        [pos_pairs[:, 0], pos_pairs[:, 1], neg_pairs[:, 0], neg_pairs[:, 1]]
    )
    rel_idx = rels[:, 0]
    ent_flat, rel_rows = _sc_gather(ent_idx, rel_idx,
                                    ent_embs.reshape(-1), rel_emb)
    out = _dense_loss(ent_flat.reshape(4 * _B, _DIM), rel_rows)
    return out[0, 0]


# speedup vs baseline: 1.6253x; 1.6253x over previous
import functools

import jax
import jax.numpy as jnp
from jax import lax
from jax.experimental import pallas as pl
from jax.experimental.pallas import tpu as pltpu
from jax.experimental.pallas import tpu_sc as plsc

_B = 16384
_DIM = 64
_MARGIN = 1.0

_NC = 2
_NS = 16
_NW = _NC * _NS

_CHUNK = 512
_GRP = 16


def _sc_gather(ent_idx, rel_idx, ent_embs, rel_emb):
    n_ent_rows = ent_idx.shape[0]
    n_rel_rows = rel_idx.shape[0]
    ent_per_w = n_ent_rows // _NW
    rel_per_w = n_rel_rows // _NW
    mesh = plsc.VectorSubcoreMesh(core_axis_name="c", subcore_axis_name="s")

    @functools.partial(
        pl.kernel,
        out_type=(
            jax.ShapeDtypeStruct((n_ent_rows, _DIM), jnp.float32),
            jax.ShapeDtypeStruct((n_rel_rows, _DIM), jnp.float32),
        ),
        mesh=mesh,
        scratch_types=[
            pltpu.VMEM((_CHUNK,), jnp.int32),
            pltpu.VMEM((_CHUNK, _DIM), jnp.float32),
            pltpu.SemaphoreType.DMA,
        ],
    )
    def gather_kernel(ent_idx_hbm, rel_idx_hbm, ent_hbm, rel_hbm,
                      ent_out, rel_out, idx_v, rows_v, sem):
        wid = lax.axis_index("s") * _NC + lax.axis_index("c")

        def do_table(idx_hbm, tab_hbm, out_hbm, per_w):
            base = wid * per_w

            def chunk_body(c, carry):
                off = base + c * _CHUNK
                pltpu.sync_copy(idx_hbm.at[pl.ds(off, _CHUNK)], idx_v)

                def grp_fire(g, carry):
                    gbase = pl.multiple_of(g * _GRP, _GRP)
                    vec = idx_v[pl.ds(gbase, _GRP)]
                    for l in range(_GRP):
                        pltpu.make_async_copy(
                            tab_hbm.at[pl.ds(vec[l], 1), :],
                            rows_v.at[pl.ds(gbase + l, 1), :],
                            sem,
                        ).start()
                    return carry

                lax.fori_loop(0, _CHUNK // _GRP, grp_fire, jnp.int32(0))

                def grp_drain(g, carry):
                    pltpu.make_async_copy(
                        tab_hbm.at[pl.ds(0, 1), :],
                        rows_v.at[pl.ds(0, 1), :],
                        sem,
                    ).wait()
                    return carry

                lax.fori_loop(0, _CHUNK, grp_drain, jnp.int32(0))
                pltpu.sync_copy(rows_v, out_hbm.at[pl.ds(off, _CHUNK)])
                return carry

            lax.fori_loop(0, per_w // _CHUNK, chunk_body, jnp.int32(0))

        do_table(ent_idx_hbm, ent_hbm, ent_out, ent_per_w)
        do_table(rel_idx_hbm, rel_hbm, rel_out, rel_per_w)

    return gather_kernel(ent_idx, rel_idx, ent_embs, rel_emb)


_BLK = 2048


def _dense_body(hp_ref, tp_ref, hn_ref, tn_ref, r_ref, out_ref):
    i = pl.program_id(0)

    def nrm(x):
        n = jnp.sqrt(jnp.sum(x * x, axis=1, keepdims=True))
        return x / jnp.maximum(n, 1e-12)

    hp = nrm(hp_ref[...])
    tp = nrm(tp_ref[...])
    hn = nrm(hn_ref[...])
    tn = nrm(tn_ref[...])
    r = nrm(r_ref[...])
    pos_e = jnp.sqrt(jnp.sum((hp + r - tp) ** 2, axis=1))
    neg_e = jnp.sqrt(jnp.sum((hn + r - tn) ** 2, axis=1))
    loss = jnp.maximum(_MARGIN + pos_e - neg_e, 0.0)
    s = jnp.sum(loss)

    @pl.when(i == 0)
    def _init():
        out_ref[0, 0] = s

    @pl.when(i != 0)
    def _acc():
        out_ref[0, 0] += s

    @pl.when(i == pl.num_programs(0) - 1)
    def _final():
        out_ref[0, 0] = out_ref[0, 0] / _B


def _dense_loss(ent_rows, rel_rows):
    grid = _B // _BLK
    nblk = grid

    def section(k):
        return pl.BlockSpec((_BLK, _DIM), lambda i, k=k: (k * nblk + i, 0))

    return pl.pallas_call(
        _dense_body,
        grid=(grid,),
        in_specs=[section(0), section(1), section(2), section(3),
                  pl.BlockSpec((_BLK, _DIM), lambda i: (i, 0))],
        out_specs=pl.BlockSpec((1, 1), lambda i: (0, 0),
                               memory_space=pltpu.SMEM),
        out_shape=jax.ShapeDtypeStruct((1, 1), jnp.float32),
    )(ent_rows, ent_rows, ent_rows, ent_rows, rel_rows)


def kernel(pos_pairs, neg_pairs, rels, ent_embs, alignments, rel_emb):
    ent_idx = jnp.concatenate(
        [pos_pairs[:, 0], pos_pairs[:, 1], neg_pairs[:, 0], neg_pairs[:, 1]]
    )
    rel_idx = rels[:, 0]
    ent_rows, rel_rows = _sc_gather(ent_idx, rel_idx, ent_embs, rel_emb)
    out = _dense_loss(ent_rows, rel_rows)
    return out[0, 0]


# speedup vs baseline: 1.6275x; 1.0013x over previous
import functools

import jax
import jax.numpy as jnp
from jax import lax
from jax.experimental import pallas as pl
from jax.experimental.pallas import tpu as pltpu
from jax.experimental.pallas import tpu_sc as plsc

_B = 16384
_DIM = 64
_MARGIN = 1.0

_NC = 2
_NS = 16
_NW = _NC * _NS

_CHUNK = 512
_GRP = 16


def _sc_gather(ent_idx, rel_idx, ent_embs, rel_emb):
    n_ent_rows = ent_idx.shape[0]
    n_rel_rows = rel_idx.shape[0]
    ent_per_w = n_ent_rows // _NW
    rel_per_w = n_rel_rows // _NW
    mesh = plsc.VectorSubcoreMesh(core_axis_name="c", subcore_axis_name="s")

    @functools.partial(
        pl.kernel,
        out_type=(
            jax.ShapeDtypeStruct((n_ent_rows, _DIM), jnp.float32),
            jax.ShapeDtypeStruct((n_rel_rows, _DIM), jnp.float32),
        ),
        mesh=mesh,
        scratch_types=[
            pltpu.VMEM((_CHUNK,), jnp.int32),
            pltpu.VMEM((_CHUNK, _DIM), jnp.float32),
            pltpu.SemaphoreType.DMA,
        ],
    )
    def gather_kernel(ent_idx_hbm, rel_idx_hbm, ent_hbm, rel_hbm,
                      ent_out, rel_out, idx_v, rows_v, sem):
        wid = lax.axis_index("s") * _NC + lax.axis_index("c")

        def do_table(idx_hbm, tab_hbm, out_hbm, per_w, chunk, idx_v, rows_v):
            base = wid * per_w

            def chunk_body(c, carry):
                off = base + c * chunk
                pltpu.sync_copy(idx_hbm.at[pl.ds(off, chunk)], idx_v)

                def grp_fire(g, carry):
                    gbase = pl.multiple_of(g * _GRP, _GRP)
                    vec = idx_v[pl.ds(gbase, _GRP)]
                    for l in range(_GRP):
                        pltpu.make_async_copy(
                            tab_hbm.at[pl.ds(vec[l], 1), :],
                            rows_v.at[pl.ds(gbase + l, 1), :],
                            sem,
                        ).start()
                    return carry

                lax.fori_loop(0, chunk // _GRP, grp_fire, jnp.int32(0))

                def grp_drain(g, carry):
                    pltpu.make_async_copy(
                        tab_hbm.at[pl.ds(0, 1), :],
                        rows_v.at[pl.ds(0, 1), :],
                        sem,
                    ).wait()
                    return carry

                lax.fori_loop(0, chunk, grp_drain, jnp.int32(0))
                pltpu.sync_copy(rows_v, out_hbm.at[pl.ds(off, chunk)])
                return carry

            lax.fori_loop(0, per_w // chunk, chunk_body, jnp.int32(0))

        do_table(ent_idx_hbm, ent_hbm, ent_out, ent_per_w, _CHUNK,
                 idx_v, rows_v)
        do_table(rel_idx_hbm, rel_hbm, rel_out, rel_per_w, _CHUNK,
                 idx_v, rows_v)

    return gather_kernel(ent_idx, rel_idx, ent_embs, rel_emb)


_BLK = 2048


def _dense_body(hp_ref, tp_ref, hn_ref, tn_ref, r_ref, out_ref):
    i = pl.program_id(0)

    def nrm(x):
        n = jnp.sqrt(jnp.sum(x * x, axis=1, keepdims=True))
        return x / jnp.maximum(n, 1e-12)

    hp = nrm(hp_ref[...])
    tp = nrm(tp_ref[...])
    hn = nrm(hn_ref[...])
    tn = nrm(tn_ref[...])
    r = nrm(r_ref[...])
    pos_e = jnp.sqrt(jnp.sum((hp + r - tp) ** 2, axis=1))
    neg_e = jnp.sqrt(jnp.sum((hn + r - tn) ** 2, axis=1))
    loss = jnp.maximum(_MARGIN + pos_e - neg_e, 0.0)
    s = jnp.sum(loss)

    @pl.when(i == 0)
    def _init():
        out_ref[0, 0] = s

    @pl.when(i != 0)
    def _acc():
        out_ref[0, 0] += s

    @pl.when(i == pl.num_programs(0) - 1)
    def _final():
        out_ref[0, 0] = out_ref[0, 0] / _B


def _dense_loss(ent_rows, rel_rows):
    grid = _B // _BLK
    nblk = grid

    def section(k):
        return pl.BlockSpec((_BLK, _DIM), lambda i, k=k: (k * nblk + i, 0))

    return pl.pallas_call(
        _dense_body,
        grid=(grid,),
        in_specs=[section(0), section(1), section(2), section(3),
                  pl.BlockSpec((_BLK, _DIM), lambda i: (i, 0))],
        out_specs=pl.BlockSpec((1, 1), lambda i: (0, 0),
                               memory_space=pltpu.SMEM),
        out_shape=jax.ShapeDtypeStruct((1, 1), jnp.float32),
    )(ent_rows, ent_rows, ent_rows, ent_rows, rel_rows)


def kernel(pos_pairs, neg_pairs, rels, ent_embs, alignments, rel_emb):
    ent_idx = jnp.concatenate(
        [pos_pairs[:, 0], pos_pairs[:, 1], neg_pairs[:, 0], neg_pairs[:, 1]]
    )
    rel_idx = rels[:, 0]
    ent_rows, rel_rows = _sc_gather(ent_idx, rel_idx, ent_embs, rel_emb)
    out = _dense_loss(ent_rows, rel_rows)
    return out[0, 0]
